# baseline (speedup 1.0000x reference)
import jax
import jax.numpy as jnp
from jax.experimental import pallas as pl
from jax.experimental.pallas import tpu as pltpu

S = 32768
A = 1024
N = 256
PREV0 = S - 2 * N
TERM0 = S - N


def _body(x_ref, d_ref, o_ref, buf, sin, sout):
    cin = pltpu.make_async_copy(x_ref.at[pl.ds(PREV0, N), :], buf, sin)
    cin.start()
    cin.wait()
    buf[...] = buf[...] + d_ref[...]
    cout = pltpu.make_async_copy(buf, o_ref.at[pl.ds(TERM0, N), :], sout)
    cout.start()
    cout.wait()


def kernel(expected, drift):
    drift2d = drift.reshape(N, 1)
    return pl.pallas_call(
        _body,
        in_specs=[
            pl.BlockSpec(memory_space=pltpu.MemorySpace.HBM),
            pl.BlockSpec(memory_space=pltpu.MemorySpace.VMEM),
        ],
        out_specs=pl.BlockSpec(memory_space=pltpu.MemorySpace.HBM),
        out_shape=jax.ShapeDtypeStruct((S, A), expected.dtype),
        input_output_aliases={0: 0},
        scratch_shapes=[
            pltpu.VMEM((N, A), jnp.float32),
            pltpu.SemaphoreType.DMA,
            pltpu.SemaphoreType.DMA,
        ],
    )(expected, drift2d)


# speedup vs baseline: 1.0375x; 1.0375x over previous
import jax
import jax.numpy as jnp
from jax.experimental import pallas as pl
from jax.experimental.pallas import tpu as pltpu

S = 32768
A = 1024
N = 256
B = 2048
M = 5
LAG = 0
NCH = S // B


def _body(x_ref, d_ref, o_ref, *rest):
    bufs = rest[:M]
    isem, osem = rest[M], rest[M + 1]
    cins = [None] * NCH
    couts = [None] * NCH
    waited = set()

    def start_in(i):
        b = i % M
        c = pltpu.make_async_copy(
            x_ref.at[pl.ds(i * B, B), :], bufs[b], isem.at[b])
        c.start()
        cins[i] = c

    for i in range(M):
        start_in(i)
    for i in range(NCH):
        b = i % M
        cins[i].wait()
        if i == NCH - 1:
            bufs[b][B - N:B, :] = bufs[b][B - 2 * N:B - N, :] + d_ref[...]
        c = pltpu.make_async_copy(
            bufs[b], o_ref.at[pl.ds(i * B, B), :], osem.at[b])
        c.start()
        couts[i] = c
        j = i - LAG
        if j >= 0 and j + M < NCH:
            couts[j].wait()
            waited.add(j)
            start_in(j + M)
    for i in range(NCH):
        if i not in waited:
            couts[i].wait()


def kernel(expected, drift):
    drift2d = drift.reshape(N, 1)
    return pl.pallas_call(
        _body,
        in_specs=[
            pl.BlockSpec(memory_space=pltpu.MemorySpace.HBM),
            pl.BlockSpec(memory_space=pltpu.MemorySpace.VMEM),
        ],
        out_specs=pl.BlockSpec(memory_space=pltpu.MemorySpace.HBM),
        out_shape=jax.ShapeDtypeStruct((S, A), expected.dtype),
        scratch_shapes=(
            [pltpu.VMEM((B, A), jnp.float32) for _ in range(M)]
            + [pltpu.SemaphoreType.DMA((M,)), pltpu.SemaphoreType.DMA((M,))]
        ),
    )(expected, drift2d)
